# Initial kernel scaffold; baseline (speedup 1.0000x reference)
import jax
import jax.numpy as jnp
from jax.experimental import pallas as pl


def kernel(adj_indices, adj_values, embeds):
    raise NotImplementedError("write your pallas kernel here")


# speedup vs baseline: 3.2125x; 3.2125x over previous
import functools
import jax
import jax.numpy as jnp
from jax import lax
from jax.experimental import pallas as pl
from jax.experimental.pallas import tpu as pltpu
from jax.experimental.pallas import tpu_sc as plsc

L = 16
NC = 2
NS = 16
C = 128


@functools.partial(jax.jit, static_argnums=(0, 1, 2, 3, 4))
def _spmm(N, D, NCHUNK, RPT, R_LAST, emb0, emb1, srcs, dsts, vals, zrows):
    DH = D // NC
    NP = NS * RPT

    mesh = plsc.VectorSubcoreMesh(
        core_axis_name="c", subcore_axis_name="s", num_cores=NC, num_subcores=NS
    )

    @functools.partial(
        pl.kernel,
        out_type=jax.ShapeDtypeStruct((NC, N, D // NC), jnp.float32),
        mesh=mesh,
        compiler_params=pltpu.CompilerParams(use_tc_tiling_on_sc=False),
        scratch_types=[
            pltpu.VMEM_SHARED((NP, DH), jnp.float32),
            pltpu.VMEM((NCHUNK, C), jnp.int32),
            pltpu.VMEM((NCHUNK, C), jnp.int32),
            pltpu.VMEM((NCHUNK, C), jnp.float32),
            pltpu.VMEM((C, DH), jnp.float32),
            pltpu.SemaphoreType.DMA,
        ],
    )
    def run(emb0_h, emb1_h, srcs_h, dsts_h, vals_h, zrows_h, out_h,
            acc, src_v, dst_v, val_v, gbuf, gsem):
        c = lax.axis_index("c")
        s = lax.axis_index("s")

        pltpu.sync_copy(srcs_h.at[s], src_v)
        pltpu.sync_copy(dsts_h.at[s], dst_v)
        pltpu.sync_copy(vals_h.at[s], val_v)
        pltpu.sync_copy(zrows_h, acc.at[pl.ds(s * RPT, RPT)])
        plsc.subcore_barrier()

        def main(emb_h):
            def chunk(j, carry):
                pltpu.async_copy(emb_h.at[src_v.at[j]], gbuf, gsem).wait()

                def group(g, carry2):
                    e0 = g * L
                    vv = val_v[j, pl.ds(e0, L)]
                    for i in range(L):
                        v = lax.broadcast(vv[i], (L,))
                        for k in range(DH // L):
                            sl = pl.ds(k * L, L)
                            gbuf[e0 + i, sl] = gbuf[e0 + i, sl] * v
                    return carry2

                lax.fori_loop(0, C // L, group, 0)
                pltpu.sync_copy(gbuf, acc.at[dst_v.at[j]], add=True)
                return carry

            lax.fori_loop(0, NCHUNK, chunk, 0)

        @pl.when(c == 0)
        def _():
            main(emb0_h)

        @pl.when(c == 1)
        def _():
            main(emb1_h)

        plsc.subcore_barrier()

        r0 = s * RPT

        @pl.when(s < NS - 1)
        def _():
            pltpu.sync_copy(acc.at[pl.ds(r0, RPT)],
                            out_h.at[c, pl.ds(r0, RPT)])

        @pl.when(s == NS - 1)
        def _():
            pltpu.sync_copy(acc.at[pl.ds(r0, R_LAST)],
                            out_h.at[c, pl.ds(r0, R_LAST)])

    return run(emb0, emb1, srcs, dsts, vals, zrows)


def kernel(adj_indices, adj_values, embeds):
    N, D = embeds.shape
    E = adj_values.shape[0]
    DH = D // NC

    EPT_RAW = -(-E // (NS * C)) * C
    EP = EPT_RAW * NS
    pad = EP - E
    src = adj_indices[1]
    dst = adj_indices[0]
    val = adj_values
    if pad:
        zi = jnp.zeros((pad,), jnp.int32)
        src = jnp.concatenate([src, zi])
        dst = jnp.concatenate([dst, zi])
        val = jnp.concatenate([val, jnp.zeros((pad,), jnp.float32)])
    NCHUNK = EPT_RAW // C

    srcs = src.reshape(NS, NCHUNK, C)
    dsts = dst.reshape(NS, NCHUNK, C)
    vals = val.reshape(NS, NCHUNK, C)

    RPT = (-(-N // NS) + 7) // 8 * 8
    R_LAST = N - (NS - 1) * RPT

    emb0 = embeds[:, :DH]
    emb1 = embeds[:, DH:]
    zrows = jnp.zeros((RPT, DH), jnp.float32)

    halves = _spmm(N, D, NCHUNK, RPT, R_LAST, emb0, emb1, srcs, dsts, vals, zrows)
    return halves.transpose(1, 0, 2).reshape(N, D)


# speedup vs baseline: 4.1242x; 1.2838x over previous
import functools
import jax
import jax.numpy as jnp
from jax import lax
from jax.experimental import pallas as pl
from jax.experimental.pallas import tpu as pltpu
from jax.experimental.pallas import tpu_sc as plsc

L = 16
NC = 2
NS = 16
C = 128


@functools.partial(jax.jit, static_argnums=(0, 1, 2, 3, 4))
def _spmm(N, D, NCHUNK, RPT, R_LAST, emb0, emb1, srcs, dsts, vals, zrows):
    DH = D // NC
    NP = NS * RPT

    mesh = plsc.VectorSubcoreMesh(
        core_axis_name="c", subcore_axis_name="s", num_cores=NC, num_subcores=NS
    )

    @functools.partial(
        pl.kernel,
        out_type=jax.ShapeDtypeStruct((NC, N, D // NC), jnp.float32),
        mesh=mesh,
        compiler_params=pltpu.CompilerParams(use_tc_tiling_on_sc=False),
        scratch_types=[
            pltpu.VMEM_SHARED((NP, DH), jnp.float32),
            pltpu.VMEM((NCHUNK, C), jnp.int32),
            pltpu.VMEM((NCHUNK, C), jnp.int32),
            pltpu.VMEM((NCHUNK, C), jnp.float32),
            pltpu.VMEM((C, DH), jnp.float32),
            pltpu.VMEM((C, DH), jnp.float32),
            pltpu.SemaphoreType.DMA,
            pltpu.SemaphoreType.DMA,
        ],
    )
    def run(emb0_h, emb1_h, srcs_h, dsts_h, vals_h, zrows_h, out_h,
            acc, src_v, dst_v, val_v, gbuf0, gbuf1, gsem0, gsem1):
        gbuf = (gbuf0, gbuf1)
        gsem = (gsem0, gsem1)
        c = lax.axis_index("c")
        s = lax.axis_index("s")

        pltpu.sync_copy(srcs_h.at[s], src_v)
        pltpu.sync_copy(dsts_h.at[s], dst_v)
        pltpu.sync_copy(vals_h.at[s], val_v)
        pltpu.sync_copy(zrows_h, acc.at[pl.ds(s * RPT, RPT)])
        plsc.subcore_barrier()

        def main(emb_h):
            pltpu.async_copy(emb_h.at[src_v.at[0]], gbuf[0], gsem[0])
            pltpu.async_copy(emb_h.at[src_v.at[1]], gbuf[1], gsem[1])

            def pair(t, carry):
                for b in range(2):
                    j = 2 * t + b
                    buf = gbuf[b]
                    pltpu.make_async_copy(emb_h.at[src_v.at[j]], buf,
                                          gsem[b]).wait()

                    def group(g, carry2):
                        e0 = g * L
                        vv = val_v[j, pl.ds(e0, L)]
                        for i in range(L):
                            v = lax.broadcast(vv[i], (L,))
                            for k in range(DH // L):
                                sl = pl.ds(k * L, L)
                                buf[e0 + i, sl] = buf[e0 + i, sl] * v
                        return carry2

                    lax.fori_loop(0, C // L, group, 0)
                    pltpu.sync_copy(buf, acc.at[dst_v.at[j]], add=True)

                    @pl.when(j + 2 < NCHUNK)
                    def _():
                        pltpu.async_copy(emb_h.at[src_v.at[j + 2]], buf,
                                         gsem[b])
                return carry

            lax.fori_loop(0, NCHUNK // 2, pair, 0)

        @pl.when(c == 0)
        def _():
            main(emb0_h)

        @pl.when(c == 1)
        def _():
            main(emb1_h)

        plsc.subcore_barrier()

        r0 = s * RPT

        @pl.when(s < NS - 1)
        def _():
            pltpu.sync_copy(acc.at[pl.ds(r0, RPT)],
                            out_h.at[c, pl.ds(r0, RPT)])

        @pl.when(s == NS - 1)
        def _():
            pltpu.sync_copy(acc.at[pl.ds(r0, R_LAST)],
                            out_h.at[c, pl.ds(r0, R_LAST)])

    return run(emb0, emb1, srcs, dsts, vals, zrows)


def kernel(adj_indices, adj_values, embeds):
    N, D = embeds.shape
    E = adj_values.shape[0]
    DH = D // NC

    EPT_RAW = -(-E // (NS * 2 * C)) * 2 * C
    EP = EPT_RAW * NS
    pad = EP - E
    src = adj_indices[1]
    dst = adj_indices[0]
    val = adj_values
    if pad:
        zi = jnp.zeros((pad,), jnp.int32)
        src = jnp.concatenate([src, zi])
        dst = jnp.concatenate([dst, zi])
        val = jnp.concatenate([val, jnp.zeros((pad,), jnp.float32)])
    NCHUNK = EPT_RAW // C

    srcs = src.reshape(NS, NCHUNK, C)
    dsts = dst.reshape(NS, NCHUNK, C)
    vals = val.reshape(NS, NCHUNK, C)

    RPT = (-(-N // NS) + 7) // 8 * 8
    R_LAST = N - (NS - 1) * RPT

    emb0 = embeds[:, :DH]
    emb1 = embeds[:, DH:]
    zrows = jnp.zeros((RPT, DH), jnp.float32)

    halves = _spmm(N, D, NCHUNK, RPT, R_LAST, emb0, emb1, srcs, dsts, vals, zrows)
    return halves.transpose(1, 0, 2).reshape(N, D)


# speedup vs baseline: 4.5421x; 1.1013x over previous
import functools
import jax
import jax.numpy as jnp
from jax import lax
from jax.experimental import pallas as pl
from jax.experimental.pallas import tpu as pltpu
from jax.experimental.pallas import tpu_sc as plsc

L = 16
NC = 2
NS = 16
C = 64


@functools.partial(jax.jit, static_argnums=(0, 1, 2, 3, 4))
def _spmm(N, D, NCHUNK, RPT, R_LAST, emb0, emb1, srcs, dsts, vals, zrows):
    DH = D // NC
    NP = NS * RPT

    mesh = plsc.VectorSubcoreMesh(
        core_axis_name="c", subcore_axis_name="s", num_cores=NC, num_subcores=NS
    )

    @functools.partial(
        pl.kernel,
        out_type=jax.ShapeDtypeStruct((NC, N, D // NC), jnp.float32),
        mesh=mesh,
        compiler_params=pltpu.CompilerParams(use_tc_tiling_on_sc=False),
        scratch_types=[
            pltpu.VMEM_SHARED((NP, DH), jnp.float32),
            pltpu.VMEM((NCHUNK, C), jnp.int32),
            pltpu.VMEM((NCHUNK, C), jnp.int32),
            pltpu.VMEM((NCHUNK, C), jnp.float32),
            pltpu.VMEM((C, DH), jnp.float32),
            pltpu.VMEM((C, DH), jnp.float32),
            pltpu.VMEM((C, DH), jnp.float32),
            pltpu.VMEM((C, DH), jnp.float32),
            pltpu.SemaphoreType.DMA,
            pltpu.SemaphoreType.DMA,
            pltpu.SemaphoreType.DMA,
            pltpu.SemaphoreType.DMA,
            pltpu.SemaphoreType.DMA,
            pltpu.SemaphoreType.DMA,
            pltpu.SemaphoreType.DMA,
            pltpu.SemaphoreType.DMA,
        ],
    )
    def run(emb0_h, emb1_h, srcs_h, dsts_h, vals_h, zrows_h, out_h,
            acc, src_v, dst_v, val_v,
            gbuf0, gbuf1, gbuf2, gbuf3,
            gsem0, gsem1, gsem2, gsem3,
            ssem0, ssem1, ssem2, ssem3):
        gbuf = (gbuf0, gbuf1, gbuf2, gbuf3)
        gsem = (gsem0, gsem1, gsem2, gsem3)
        ssem = (ssem0, ssem1, ssem2, ssem3)
        c = lax.axis_index("c")
        s = lax.axis_index("s")

        pltpu.sync_copy(srcs_h.at[s], src_v)
        pltpu.sync_copy(dsts_h.at[s], dst_v)
        pltpu.sync_copy(vals_h.at[s], val_v)
        pltpu.sync_copy(zrows_h, acc.at[pl.ds(s * RPT, RPT)])
        plsc.subcore_barrier()

        def main(emb_h):
            pltpu.async_copy(emb_h.at[src_v.at[0]], gbuf[0], gsem[0])
            pltpu.async_copy(emb_h.at[src_v.at[1]], gbuf[1], gsem[1])

            def quad(t, carry):
                for b in range(4):
                    j = 4 * t + b
                    buf = gbuf[b]
                    bn = (b + 2) % 4

                    @pl.when(j >= 2)
                    def _():
                        pltpu.make_async_copy(gbuf[bn], acc.at[dst_v.at[j]],
                                              ssem[bn]).wait()

                    @pl.when(j + 2 < NCHUNK)
                    def _():
                        pltpu.async_copy(emb_h.at[src_v.at[j + 2]], gbuf[bn],
                                         gsem[bn])

                    pltpu.make_async_copy(emb_h.at[src_v.at[j]], buf,
                                          gsem[b]).wait()

                    def group(g, carry2):
                        e0 = g * L
                        vv = val_v[j, pl.ds(e0, L)]
                        for i in range(L):
                            v = lax.broadcast(vv[i], (L,))
                            for k in range(DH // L):
                                sl = pl.ds(k * L, L)
                                buf[e0 + i, sl] = buf[e0 + i, sl] * v
                        return carry2

                    lax.fori_loop(0, C // L, group, 0)
                    pltpu.async_copy(buf, acc.at[dst_v.at[j]], ssem[b],
                                     add=True)
                return carry

            lax.fori_loop(0, NCHUNK // 4, quad, 0)
            pltpu.make_async_copy(gbuf[2], acc.at[dst_v.at[NCHUNK - 2]],
                                  ssem[2]).wait()
            pltpu.make_async_copy(gbuf[3], acc.at[dst_v.at[NCHUNK - 1]],
                                  ssem[3]).wait()

        @pl.when(c == 0)
        def _():
            main(emb0_h)

        @pl.when(c == 1)
        def _():
            main(emb1_h)

        plsc.subcore_barrier()

        r0 = s * RPT

        @pl.when(s < NS - 1)
        def _():
            pltpu.sync_copy(acc.at[pl.ds(r0, RPT)],
                            out_h.at[c, pl.ds(r0, RPT)])

        @pl.when(s == NS - 1)
        def _():
            pltpu.sync_copy(acc.at[pl.ds(r0, R_LAST)],
                            out_h.at[c, pl.ds(r0, R_LAST)])

    return run(emb0, emb1, srcs, dsts, vals, zrows)


def kernel(adj_indices, adj_values, embeds):
    N, D = embeds.shape
    E = adj_values.shape[0]
    DH = D // NC

    EPT_RAW = -(-E // (NS * 4 * C)) * 4 * C
    EP = EPT_RAW * NS
    pad = EP - E
    src = adj_indices[1]
    dst = adj_indices[0]
    val = adj_values
    if pad:
        zi = jnp.zeros((pad,), jnp.int32)
        src = jnp.concatenate([src, zi])
        dst = jnp.concatenate([dst, zi])
        val = jnp.concatenate([val, jnp.zeros((pad,), jnp.float32)])
    NCHUNK = EPT_RAW // C

    srcs = src.reshape(NS, NCHUNK, C)
    dsts = dst.reshape(NS, NCHUNK, C)
    vals = val.reshape(NS, NCHUNK, C)

    RPT = (-(-N // NS) + 7) // 8 * 8
    R_LAST = N - (NS - 1) * RPT

    emb0 = embeds[:, :DH]
    emb1 = embeds[:, DH:]
    zrows = jnp.zeros((RPT, DH), jnp.float32)

    halves = _spmm(N, D, NCHUNK, RPT, R_LAST, emb0, emb1, srcs, dsts, vals, zrows)
    return halves.transpose(1, 0, 2).reshape(N, D)


# speedup vs baseline: 7.6336x; 1.6806x over previous
import functools
import jax
import jax.numpy as jnp
from jax import lax
from jax.experimental import pallas as pl
from jax.experimental.pallas import tpu as pltpu
from jax.experimental.pallas import tpu_sc as plsc

L = 16
NC = 2
NS = 16
C = 64


@functools.partial(jax.jit, static_argnums=(0, 1, 2, 3, 4))
def _spmm(N, D, NCHUNK, RPT, R_LAST, emb0, emb1, srcs, dsts, vals, zrows):
    DH = D // NC
    NP = NS * RPT

    mesh = plsc.VectorSubcoreMesh(
        core_axis_name="c", subcore_axis_name="s", num_cores=NC, num_subcores=NS
    )

    @functools.partial(
        pl.kernel,
        out_type=jax.ShapeDtypeStruct((NC, N, D // NC), jnp.float32),
        mesh=mesh,
        compiler_params=pltpu.CompilerParams(use_tc_tiling_on_sc=False),
        scratch_types=[
            pltpu.VMEM_SHARED((NP, DH), jnp.float32),
            pltpu.VMEM((NCHUNK, C), jnp.int32),
            pltpu.VMEM((NCHUNK, C), jnp.int32),
            pltpu.VMEM((NCHUNK, C), jnp.float32),
            pltpu.VMEM((C, DH), jnp.float32),
            pltpu.VMEM((C, DH), jnp.float32),
            pltpu.VMEM((C, DH), jnp.float32),
            pltpu.VMEM((C, DH), jnp.float32),
            pltpu.SemaphoreType.DMA,
            pltpu.SemaphoreType.DMA,
            pltpu.SemaphoreType.DMA,
            pltpu.SemaphoreType.DMA,
            pltpu.SemaphoreType.DMA,
            pltpu.SemaphoreType.DMA,
            pltpu.SemaphoreType.DMA,
            pltpu.SemaphoreType.DMA,
        ],
    )
    def run(emb0_h, emb1_h, srcs_h, dsts_h, vals_h, zrows_h, out_h,
            acc, src_v, dst_v, val_v,
            gbuf0, gbuf1, gbuf2, gbuf3,
            gsem0, gsem1, gsem2, gsem3,
            ssem0, ssem1, ssem2, ssem3):
        gbuf = (gbuf0, gbuf1, gbuf2, gbuf3)
        gsem = (gsem0, gsem1, gsem2, gsem3)
        ssem = (ssem0, ssem1, ssem2, ssem3)
        c = lax.axis_index("c")
        s = lax.axis_index("s")

        pltpu.sync_copy(srcs_h.at[s], src_v)
        pltpu.sync_copy(dsts_h.at[s], dst_v)
        pltpu.sync_copy(vals_h.at[s], val_v)
        pltpu.sync_copy(zrows_h, acc.at[pl.ds(s * RPT, RPT)])
        plsc.subcore_barrier()

        def main(emb_h):
            pltpu.async_copy(emb_h.at[src_v.at[0]], gbuf[0], gsem[0])
            pltpu.async_copy(emb_h.at[src_v.at[1]], gbuf[1], gsem[1])

            def quad(t, carry):
                for b in range(4):
                    j = 4 * t + b
                    buf = gbuf[b]
                    bn = (b + 2) % 4

                    @pl.when(j >= 2)
                    def _():
                        pltpu.make_async_copy(gbuf[bn], acc.at[dst_v.at[j]],
                                              ssem[bn]).wait()

                    @pl.when(j + 2 < NCHUNK)
                    def _():
                        pltpu.async_copy(emb_h.at[src_v.at[j + 2]], gbuf[bn],
                                         gsem[bn])

                    pltpu.make_async_copy(emb_h.at[src_v.at[j]], buf,
                                          gsem[b]).wait()

                    @plsc.parallel_loop(0, C // L, 1, unroll=2)
                    def group(g):
                        e0 = g * L
                        vv = val_v[j, pl.ds(e0, L)]
                        for i in range(L):
                            v = lax.broadcast(vv[i], (L,))
                            for k in range(DH // L):
                                sl = pl.ds(k * L, L)
                                buf[e0 + i, sl] = buf[e0 + i, sl] * v
                    pltpu.async_copy(buf, acc.at[dst_v.at[j]], ssem[b],
                                     add=True)
                return carry

            lax.fori_loop(0, NCHUNK // 4, quad, 0)
            pltpu.make_async_copy(gbuf[2], acc.at[dst_v.at[NCHUNK - 2]],
                                  ssem[2]).wait()
            pltpu.make_async_copy(gbuf[3], acc.at[dst_v.at[NCHUNK - 1]],
                                  ssem[3]).wait()

        @pl.when(c == 0)
        def _():
            main(emb0_h)

        @pl.when(c == 1)
        def _():
            main(emb1_h)

        plsc.subcore_barrier()

        r0 = s * RPT

        @pl.when(s < NS - 1)
        def _():
            pltpu.sync_copy(acc.at[pl.ds(r0, RPT)],
                            out_h.at[c, pl.ds(r0, RPT)])

        @pl.when(s == NS - 1)
        def _():
            pltpu.sync_copy(acc.at[pl.ds(r0, R_LAST)],
                            out_h.at[c, pl.ds(r0, R_LAST)])

    return run(emb0, emb1, srcs, dsts, vals, zrows)


def kernel(adj_indices, adj_values, embeds):
    N, D = embeds.shape
    E = adj_values.shape[0]
    DH = D // NC

    EPT_RAW = -(-E // (NS * 4 * C)) * 4 * C
    EP = EPT_RAW * NS
    pad = EP - E
    src = adj_indices[1]
    dst = adj_indices[0]
    val = adj_values
    if pad:
        zi = jnp.zeros((pad,), jnp.int32)
        src = jnp.concatenate([src, zi])
        dst = jnp.concatenate([dst, zi])
        val = jnp.concatenate([val, jnp.zeros((pad,), jnp.float32)])
    NCHUNK = EPT_RAW // C

    srcs = src.reshape(NS, NCHUNK, C)
    dsts = dst.reshape(NS, NCHUNK, C)
    vals = val.reshape(NS, NCHUNK, C)

    RPT = (-(-N // NS) + 7) // 8 * 8
    R_LAST = N - (NS - 1) * RPT

    emb0 = embeds[:, :DH]
    emb1 = embeds[:, DH:]
    zrows = jnp.zeros((RPT, DH), jnp.float32)

    halves = _spmm(N, D, NCHUNK, RPT, R_LAST, emb0, emb1, srcs, dsts, vals, zrows)
    return halves.transpose(1, 0, 2).reshape(N, D)
